# baseline (speedup 1.0000x reference)
import functools

import jax
import jax.numpy as jnp
import numpy as np
from jax import lax
from jax.experimental import pallas as pl
from jax.experimental.pallas import tpu as pltpu
from jax.experimental.pallas import tpu_sc as plsc

D = 64
L_SEQ = 200
NUM_CORES = 2
NUM_SUBCORES = 16
NW = NUM_CORES * NUM_SUBCORES
LANES = 16
GCHUNKS = ((0, 104), (104, 96))
NBUF = 3

PERM = np.array(
    [32 * k + (2 * jp if jp < 16 else 2 * (jp - 16) + 1)
     for j in range(D)
     for k, jp in [(j // 32, j % 32)]], dtype=np.int32)


def _sc_user_gather(user_ids, user_table_bf, token):
    B = user_ids.shape[0]
    b_per_w = B // NW

    mesh = plsc.VectorSubcoreMesh(
        core_axis_name="c", subcore_axis_name="s",
        num_cores=NUM_CORES, num_subcores=NUM_SUBCORES)

    @functools.partial(
        pl.kernel,
        out_type=jax.ShapeDtypeStruct((B, D), jnp.bfloat16),
        mesh=mesh,
        compiler_params=pltpu.CompilerParams(use_tc_tiling_on_sc=False),
        scratch_types=[
            pltpu.VMEM((b_per_w,), jnp.int32),
            pltpu.VMEM((b_per_w, D), jnp.bfloat16),
            pltpu.SemaphoreType.DMA,
        ],
    )
    def u_kernel(uid_hbm, utab_hbm, tok_hbm, u_out, uidx, ubuf, usem):
        del tok_hbm
        wid = lax.axis_index("s") * NUM_CORES + lax.axis_index("c")
        base = wid * b_per_w
        pltpu.sync_copy(uid_hbm.at[pl.ds(base, b_per_w)], uidx)
        udescs = [
            pltpu.make_async_copy(
                utab_hbm.at[uidx.at[pl.ds(c * 128, 128)]],
                ubuf.at[pl.ds(c * 128, 128), :],
                usem)
            for c in range(b_per_w // 128)
        ]
        for d_ in udescs:
            d_.start()
        for d_ in udescs:
            d_.wait()
        pltpu.sync_copy(ubuf, u_out.at[pl.ds(base, b_per_w), :])

    return u_kernel(user_ids, user_table_bf, token)


def _sc_seq_pool(seq, item_table_bf):
    B = seq.shape[0]
    assert B % NW == 0
    b_per_w = B // NW
    half = 128
    nblk = b_per_w // half

    mesh = plsc.VectorSubcoreMesh(
        core_axis_name="c", subcore_axis_name="s",
        num_cores=NUM_CORES, num_subcores=NUM_SUBCORES)

    @functools.partial(
        pl.kernel,
        out_type=jax.ShapeDtypeStruct((B, D), jnp.float32),
        mesh=mesh,
        compiler_params=pltpu.CompilerParams(
            use_tc_tiling_on_sc=False, needs_layout_passes=False),
        scratch_types=[
            pltpu.VMEM((half, L_SEQ), jnp.int32),
            pltpu.VMEM((NBUF, L_SEQ, D), jnp.bfloat16),
            pltpu.VMEM((half, D), jnp.float32),
            pltpu.SemaphoreType.DMA,
        ],
    )
    def sc_kernel(seq_hbm, itab_hbm, ssum_out, idx_v, gbuf, ostage, gsem):
        wid = lax.axis_index("s") * NUM_CORES + lax.axis_index("c")
        base = wid * b_per_w

        def descs(r, slot):
            return [
                pltpu.make_async_copy(
                    itab_hbm.at[idx_v.at[r, pl.ds(off, n)]],
                    gbuf.at[slot, pl.ds(off, n), :],
                    gsem)
                for off, n in GCHUNKS
            ]

        for blk in range(nblk):
            row0 = base + blk * half
            pltpu.sync_copy(seq_hbm.at[pl.ds(row0, half), :], idx_v)
            for p in range(NBUF - 1):
                for d_ in descs(p, p):
                    d_.start()

            def row_body(r, carry):
                slot = lax.rem(r, NBUF)
                for d_ in descs(r, slot):
                    d_.wait()

                nxt = r + NBUF - 1

                @pl.when(nxt < half)
                def _():
                    for d_ in descs(nxt, lax.rem(nxt, NBUF)):
                        d_.start()

                def acc_body(l, acc):
                    out = list(acc)
                    for k in range(D // 32):
                        x = gbuf[slot, l, pl.ds(k * 32, 32)]
                        a_, b_ = plsc.unpack(
                            x, format=plsc.PackFormat.INTERLEAVED)
                        out[2 * k] = out[2 * k] + a_
                        out[2 * k + 1] = out[2 * k + 1] + b_
                    return tuple(out)

                acc = lax.fori_loop(
                    0, L_SEQ, acc_body,
                    tuple(jnp.zeros((LANES,), jnp.float32)
                          for _ in range(D // LANES)),
                    unroll=8)
                for k in range(D // LANES):
                    ostage[r, pl.ds(k * LANES, LANES)] = acc[k]
                return carry

            lax.fori_loop(0, half, row_body, 0)
            pltpu.sync_copy(ostage, ssum_out.at[pl.ds(row0, half), :])

    return sc_kernel(seq, item_table_bf)


def _mlp_kernel(u_ref, s_ref, seq_ref, slen_ref, e0_ref,
                w1a_ref, w1b_ref, w1c_ref, b1_ref, w2_ref, b2_ref, o_ref):
    seqblk = seq_ref[...]
    n0 = jnp.sum((seqblk == 0).astype(jnp.float32), axis=1, keepdims=True)
    cnt = jnp.float32(L_SEQ) - n0
    s = s_ref[...] - n0 * e0_ref[...]
    seq_vec = jnp.where(cnt > 0.0, s / (cnt + 1e-9), 0.0)
    slen = slen_ref[...].astype(jnp.float32)
    u = u_ref[...].astype(jnp.float32)
    hp = jax.lax.Precision.HIGHEST
    h = (jnp.dot(u, w1a_ref[...], precision=hp)
         + jnp.dot(seq_vec, w1b_ref[...], precision=hp)
         + slen * w1c_ref[...] + b1_ref[...])
    h = jnp.maximum(h, 0.0)
    o_ref[...] = (lax.dot_general(w2_ref[...], h, (((0,), (1,)), ((), ())),
                                  precision=hp)
                  + b2_ref[...])


def kernel(user_ids, seq, seq_len, user_table, item_table, W1, b1, W2, b2):
    B = user_ids.shape[0]
    user_ids = user_ids.astype(jnp.int32)
    item_bf = item_table.astype(jnp.bfloat16)
    user_bf = user_table.astype(jnp.bfloat16)
    ssum = _sc_seq_pool(seq, item_bf)
    token = jnp.zeros((8,), jnp.float32) + ssum[0, :8]
    u_emb = _sc_user_gather(user_ids, user_bf, token)

    perm = jnp.asarray(PERM)
    e0p = item_bf[0:1, :].astype(jnp.float32)[:, perm]
    w1a = W1[0:D, :]
    w1bp = W1[D:2 * D, :][perm, :]
    w1c = W1[2 * D:2 * D + 1, :]
    b1r = b1.reshape(1, -1)
    b2r = b2.reshape(-1, 1)
    slen = seq_len.reshape(B, 1).astype(jnp.int32)

    TB = 1024
    grid = (B // TB,)
    H = W1.shape[1]

    def row_spec(w):
        return pl.BlockSpec((TB, w), lambda i: (i, 0))

    def col_spec(hgt):
        return pl.BlockSpec((hgt, TB), lambda i: (0, i))

    def full_spec(a, b):
        return pl.BlockSpec((a, b), lambda i: (0, 0))

    out_t = pl.pallas_call(
        _mlp_kernel,
        grid=grid,
        in_specs=[
            row_spec(D), row_spec(D), row_spec(L_SEQ), row_spec(1),
            full_spec(1, D),
            full_spec(D, H), full_spec(D, H), full_spec(1, H),
            full_spec(1, H), full_spec(H, D), full_spec(D, 1),
        ],
        out_specs=col_spec(D),
        out_shape=jax.ShapeDtypeStruct((D, B), jnp.float32),
    )(u_emb, ssum, seq, slen, e0p, w1a, w1bp, w1c, b1r, W2, b2r)
    return out_t.T


# speedup vs baseline: 1.7377x; 1.7377x over previous
import functools

import jax
import jax.numpy as jnp
from jax import lax
from jax.experimental import pallas as pl
from jax.experimental.pallas import tpu as pltpu
from jax.experimental.pallas import tpu_sc as plsc

D = 64
L_SEQ = 200
NUM_CORES = 2
NUM_SUBCORES = 16
NW = NUM_CORES * NUM_SUBCORES
LANES = 16
GCHUNKS = ((0, 104), (104, 96))
NBUF = 3


def _sc_user_gather(user_ids, user_table, token):
    B = user_ids.shape[0]
    b_per_w = B // NW

    mesh = plsc.VectorSubcoreMesh(
        core_axis_name="c", subcore_axis_name="s",
        num_cores=NUM_CORES, num_subcores=NUM_SUBCORES)

    @functools.partial(
        pl.kernel,
        out_type=jax.ShapeDtypeStruct((B, D), jnp.float32),
        mesh=mesh,
        compiler_params=pltpu.CompilerParams(use_tc_tiling_on_sc=False),
        scratch_types=[
            pltpu.VMEM((b_per_w,), jnp.int32),
            pltpu.VMEM((b_per_w, D), jnp.float32),
            pltpu.SemaphoreType.DMA,
        ],
    )
    def u_kernel(uid_hbm, utab_hbm, tok_hbm, u_out, uidx, ubuf, usem):
        del tok_hbm
        wid = lax.axis_index("s") * NUM_CORES + lax.axis_index("c")
        base = wid * b_per_w
        pltpu.sync_copy(uid_hbm.at[pl.ds(base, b_per_w)], uidx)
        udescs = [
            pltpu.make_async_copy(
                utab_hbm.at[uidx.at[pl.ds(c * 128, 128)]],
                ubuf.at[pl.ds(c * 128, 128), :],
                usem)
            for c in range(b_per_w // 128)
        ]
        for d_ in udescs:
            d_.start()
        for d_ in udescs:
            d_.wait()
        pltpu.sync_copy(ubuf, u_out.at[pl.ds(base, b_per_w), :])

    return u_kernel(user_ids, user_table, token)


def _sc_seq_pool(seq, item_table):
    B = seq.shape[0]
    assert B % NW == 0
    b_per_w = B // NW
    half = 128
    nblk = b_per_w // half

    mesh = plsc.VectorSubcoreMesh(
        core_axis_name="c", subcore_axis_name="s",
        num_cores=NUM_CORES, num_subcores=NUM_SUBCORES)

    @functools.partial(
        pl.kernel,
        out_type=jax.ShapeDtypeStruct((B, D), jnp.float32),
        mesh=mesh,
        compiler_params=pltpu.CompilerParams(use_tc_tiling_on_sc=False),
        scratch_types=[
            pltpu.VMEM((half, L_SEQ), jnp.int32),
            pltpu.VMEM((NBUF, L_SEQ, D), jnp.float32),
            pltpu.VMEM((half, D), jnp.float32),
            pltpu.SemaphoreType.DMA,
        ],
    )
    def sc_kernel(seq_hbm, itab_hbm, ssum_out, idx_v, gbuf, ostage, gsem):
        wid = lax.axis_index("s") * NUM_CORES + lax.axis_index("c")
        base = wid * b_per_w

        def descs(r, slot):
            return [
                pltpu.make_async_copy(
                    itab_hbm.at[idx_v.at[r, pl.ds(off, n)]],
                    gbuf.at[slot, pl.ds(off, n), :],
                    gsem)
                for off, n in GCHUNKS
            ]

        for blk in range(nblk):
            row0 = base + blk * half
            pltpu.sync_copy(seq_hbm.at[pl.ds(row0, half), :], idx_v)
            for p in range(NBUF - 1):
                for d_ in descs(p, p):
                    d_.start()

            def row_body(r, carry):
                slot = lax.rem(r, NBUF)
                for d_ in descs(r, slot):
                    d_.wait()

                nxt = r + NBUF - 1

                @pl.when(nxt < half)
                def _():
                    for d_ in descs(nxt, lax.rem(nxt, NBUF)):
                        d_.start()

                def acc_body(l, acc):
                    return tuple(
                        acc[k] + gbuf[slot, l, pl.ds(k * LANES, LANES)]
                        for k in range(D // LANES))

                acc = lax.fori_loop(
                    0, L_SEQ, acc_body,
                    tuple(jnp.zeros((LANES,), jnp.float32)
                          for _ in range(D // LANES)),
                    unroll=8)
                for k in range(D // LANES):
                    ostage[r, pl.ds(k * LANES, LANES)] = acc[k]
                return carry

            lax.fori_loop(0, half, row_body, 0)
            pltpu.sync_copy(ostage, ssum_out.at[pl.ds(row0, half), :])

    return sc_kernel(seq, item_table)


def _mlp_kernel(u_ref, s_ref, seq_ref, slen_ref, e0_ref,
                w1a_ref, w1b_ref, w1c_ref, b1_ref, w2_ref, b2_ref, o_ref):
    seqblk = seq_ref[...]
    n0 = jnp.sum((seqblk == 0).astype(jnp.float32), axis=1, keepdims=True)
    cnt = jnp.float32(L_SEQ) - n0
    s = s_ref[...] - n0 * e0_ref[...]
    seq_vec = jnp.where(cnt > 0.0, s / (cnt + 1e-9), 0.0)
    slen = slen_ref[...].astype(jnp.float32)
    hp = jax.lax.Precision.HIGHEST
    h = (jnp.dot(u_ref[...], w1a_ref[...], precision=hp)
         + jnp.dot(seq_vec, w1b_ref[...], precision=hp)
         + slen * w1c_ref[...] + b1_ref[...])
    h = jnp.maximum(h, 0.0)
    o_ref[...] = (lax.dot_general(w2_ref[...], h, (((0,), (1,)), ((), ())),
                                  precision=hp)
                  + b2_ref[...])


def kernel(user_ids, seq, seq_len, user_table, item_table, W1, b1, W2, b2):
    B = user_ids.shape[0]
    user_ids = user_ids.astype(jnp.int32)
    ssum = _sc_seq_pool(seq, item_table)
    token = jnp.zeros((8,), jnp.float32) + ssum[0, :8]
    u_emb = _sc_user_gather(user_ids, user_table, token)

    e0 = item_table[0:1, :]
    w1a = W1[0:D, :]
    w1b = W1[D:2 * D, :]
    w1c = W1[2 * D:2 * D + 1, :]
    b1r = b1.reshape(1, -1)
    b2r = b2.reshape(-1, 1)
    slen = seq_len.reshape(B, 1).astype(jnp.int32)

    TB = 1024
    grid = (B // TB,)
    H = W1.shape[1]

    def row_spec(w):
        return pl.BlockSpec((TB, w), lambda i: (i, 0))

    def col_spec(hgt):
        return pl.BlockSpec((hgt, TB), lambda i: (0, i))

    def full_spec(a, b):
        return pl.BlockSpec((a, b), lambda i: (0, 0))

    out_t = pl.pallas_call(
        _mlp_kernel,
        grid=grid,
        in_specs=[
            row_spec(D), row_spec(D), row_spec(L_SEQ), row_spec(1),
            full_spec(1, D),
            full_spec(D, H), full_spec(D, H), full_spec(1, H),
            full_spec(1, H), full_spec(H, D), full_spec(D, 1),
        ],
        out_specs=col_spec(D),
        out_shape=jax.ShapeDtypeStruct((D, B), jnp.float32),
    )(u_emb, ssum, seq, slen, e0, w1a, w1b, w1c, b1r, W2, b2r)
    return out_t.T


# speedup vs baseline: 1.7479x; 1.0059x over previous
import functools

import jax
import jax.numpy as jnp
from jax import lax
from jax.experimental import pallas as pl
from jax.experimental.pallas import tpu as pltpu
from jax.experimental.pallas import tpu_sc as plsc

D = 64
L_SEQ = 200
NUM_CORES = 2
NUM_SUBCORES = 16
NW = NUM_CORES * NUM_SUBCORES
LANES = 16
GCHUNKS = ((0, 104), (104, 96))
NBUF = 4


def _sc_user_gather(user_ids, user_table, token):
    B = user_ids.shape[0]
    b_per_w = B // NW

    mesh = plsc.VectorSubcoreMesh(
        core_axis_name="c", subcore_axis_name="s",
        num_cores=NUM_CORES, num_subcores=NUM_SUBCORES)

    @functools.partial(
        pl.kernel,
        out_type=jax.ShapeDtypeStruct((B, D), jnp.float32),
        mesh=mesh,
        compiler_params=pltpu.CompilerParams(use_tc_tiling_on_sc=False),
        scratch_types=[
            pltpu.VMEM((b_per_w,), jnp.int32),
            pltpu.VMEM((b_per_w, D), jnp.float32),
            pltpu.SemaphoreType.DMA,
        ],
    )
    def u_kernel(uid_hbm, utab_hbm, tok_hbm, u_out, uidx, ubuf, usem):
        del tok_hbm
        wid = lax.axis_index("s") * NUM_CORES + lax.axis_index("c")
        base = wid * b_per_w
        pltpu.sync_copy(uid_hbm.at[pl.ds(base, b_per_w)], uidx)
        udescs = [
            pltpu.make_async_copy(
                utab_hbm.at[uidx.at[pl.ds(c * 128, 128)]],
                ubuf.at[pl.ds(c * 128, 128), :],
                usem)
            for c in range(b_per_w // 128)
        ]
        for d_ in udescs:
            d_.start()
        for d_ in udescs:
            d_.wait()
        pltpu.sync_copy(ubuf, u_out.at[pl.ds(base, b_per_w), :])

    return u_kernel(user_ids, user_table, token)


def _sc_seq_pool(seq, item_table):
    B = seq.shape[0]
    assert B % NW == 0
    b_per_w = B // NW
    half = 128
    nblk = b_per_w // half

    mesh = plsc.VectorSubcoreMesh(
        core_axis_name="c", subcore_axis_name="s",
        num_cores=NUM_CORES, num_subcores=NUM_SUBCORES)

    @functools.partial(
        pl.kernel,
        out_type=jax.ShapeDtypeStruct((B, D), jnp.float32),
        mesh=mesh,
        compiler_params=pltpu.CompilerParams(use_tc_tiling_on_sc=False),
        scratch_types=[
            pltpu.VMEM((half, L_SEQ), jnp.int32),
            pltpu.VMEM((NBUF, L_SEQ, D), jnp.float32),
            pltpu.VMEM((half, D), jnp.float32),
            pltpu.SemaphoreType.DMA,
            pltpu.SemaphoreType.DMA,
        ],
    )
    def sc_kernel(seq_hbm, itab_hbm, ssum_out, idx_v, gbuf, ostage,
                  gsem_a, gsem_b):
        wid = lax.axis_index("s") * NUM_CORES + lax.axis_index("c")
        base = wid * b_per_w

        def descs(r, slot):
            return [
                pltpu.make_async_copy(
                    itab_hbm.at[idx_v.at[r, pl.ds(off, n)]],
                    gbuf.at[slot, pl.ds(off, n), :],
                    sem)
                for (off, n), sem in zip(GCHUNKS, (gsem_a, gsem_b))
            ]

        def acc_rows(slot, lo, hi, acc):
            def acc_body(l, a):
                return tuple(
                    a[k] + gbuf[slot, l, pl.ds(k * LANES, LANES)]
                    for k in range(D // LANES))
            return lax.fori_loop(lo, hi, acc_body, acc, unroll=8)

        zeros4 = tuple(jnp.zeros((LANES,), jnp.float32)
                       for _ in range(D // LANES))

        for blk in range(nblk):
            row0 = base + blk * half
            pltpu.sync_copy(seq_hbm.at[pl.ds(row0, half), :], idx_v)
            for p in range(NBUF - 1):
                for d_ in descs(p, p):
                    d_.start()

            def row_body(r, carry):
                slot = lax.rem(r, NBUF)
                da, db = descs(r, slot)
                da.wait()

                nxt = r + NBUF - 1

                @pl.when(nxt < half)
                def _():
                    for d_ in descs(nxt, lax.rem(nxt, NBUF)):
                        d_.start()

                acc = acc_rows(slot, 0, GCHUNKS[0][1], zeros4)
                db.wait()
                acc = acc_rows(slot, GCHUNKS[0][1], L_SEQ, acc)
                for k in range(D // LANES):
                    ostage[r, pl.ds(k * LANES, LANES)] = acc[k]
                return carry

            lax.fori_loop(0, half, row_body, 0)
            pltpu.sync_copy(ostage, ssum_out.at[pl.ds(row0, half), :])

    return sc_kernel(seq, item_table)


def _mlp_kernel(u_ref, s_ref, seq_ref, slen_ref, e0_ref,
                w1a_ref, w1b_ref, w1c_ref, b1_ref, w2_ref, b2_ref, o_ref):
    seqblk = seq_ref[...]
    n0 = jnp.sum((seqblk == 0).astype(jnp.float32), axis=1, keepdims=True)
    cnt = jnp.float32(L_SEQ) - n0
    s = s_ref[...] - n0 * e0_ref[...]
    seq_vec = jnp.where(cnt > 0.0, s / (cnt + 1e-9), 0.0)
    slen = slen_ref[...].astype(jnp.float32)
    hp = jax.lax.Precision.HIGHEST
    h = (jnp.dot(u_ref[...], w1a_ref[...], precision=hp)
         + jnp.dot(seq_vec, w1b_ref[...], precision=hp)
         + slen * w1c_ref[...] + b1_ref[...])
    h = jnp.maximum(h, 0.0)
    o_ref[...] = (lax.dot_general(w2_ref[...], h, (((0,), (1,)), ((), ())),
                                  precision=hp)
                  + b2_ref[...])


def kernel(user_ids, seq, seq_len, user_table, item_table, W1, b1, W2, b2):
    B = user_ids.shape[0]
    user_ids = user_ids.astype(jnp.int32)
    ssum = _sc_seq_pool(seq, item_table)
    token = jnp.zeros((8,), jnp.float32) + ssum[0, :8]
    u_emb = _sc_user_gather(user_ids, user_table, token)

    e0 = item_table[0:1, :]
    w1a = W1[0:D, :]
    w1b = W1[D:2 * D, :]
    w1c = W1[2 * D:2 * D + 1, :]
    b1r = b1.reshape(1, -1)
    b2r = b2.reshape(-1, 1)
    slen = seq_len.reshape(B, 1).astype(jnp.int32)

    TB = 1024
    grid = (B // TB,)
    H = W1.shape[1]

    def row_spec(w):
        return pl.BlockSpec((TB, w), lambda i: (i, 0))

    def col_spec(hgt):
        return pl.BlockSpec((hgt, TB), lambda i: (0, i))

    def full_spec(a, b):
        return pl.BlockSpec((a, b), lambda i: (0, 0))

    out_t = pl.pallas_call(
        _mlp_kernel,
        grid=grid,
        in_specs=[
            row_spec(D), row_spec(D), row_spec(L_SEQ), row_spec(1),
            full_spec(1, D),
            full_spec(D, H), full_spec(D, H), full_spec(1, H),
            full_spec(1, H), full_spec(H, D), full_spec(D, 1),
        ],
        out_specs=col_spec(D),
        out_shape=jax.ShapeDtypeStruct((D, B), jnp.float32),
    )(u_emb, ssum, seq, slen, e0, w1a, w1b, w1c, b1r, W2, b2r)
    return out_t.T
